# initial kernel scaffold (unmeasured)
import jax
import jax.numpy as jnp
from jax import lax
from jax.experimental import pallas as pl
from jax.experimental.pallas import tpu as pltpu

N_DEV = 8
M = 1536
N = 1536
SEG = M // N_DEV


def _gelu(z):
    return 0.5 * z * (1.0 + jnp.tanh(0.7978845608 * (z + 0.044715 * z * z * z)))


def kernel(A, B):
    A16 = A.astype(jnp.bfloat16)
    B16 = B.astype(jnp.bfloat16)

    def body(a_ref, b_ref, out_ref, g_ref, rs_recv, stage, send_sems, recv_sems):
        p = lax.axis_index("i")
        b2 = (p >> 2) & 1
        b1 = (p >> 1) & 1
        b0 = p & 1
        pz = p ^ 4
        px = (p & 4) | ((p & 3) ^ 1)
        py = (p & 4) | (3 - (p & 3))

        half = b2 * 768
        quart = half + b0 * 384
        eighth = quart + b1 * SEG

        out_ref[:, :] = jnp.dot(
            a_ref[:, :], b_ref[:, :], preferred_element_type=jnp.float32
        )

        barrier = pltpu.get_barrier_semaphore()
        for nbr in (pz, px, py):
            pl.semaphore_signal(
                barrier, inc=1, device_id=(nbr,),
                device_id_type=pl.DeviceIdType.MESH,
            )
        pl.semaphore_wait(barrier, 3)

        rs_steps = (
            (0, pz, 768, (1 - b2) * 768, half, 0),
            (1, px, 384, half + (1 - b0) * 384, quart, 768),
            (2, py, SEG, quart + (1 - b1) * SEG, eighth, 1152),
        )
        for s, partner, ln, send_base, keep_base, rbase in rs_steps:
            stage[pl.ds(0, ln), :] = out_ref[pl.ds(send_base, ln), :].astype(
                jnp.bfloat16
            )
            rdma = pltpu.make_async_remote_copy(
                src_ref=stage.at[pl.ds(0, ln)],
                dst_ref=rs_recv.at[pl.ds(rbase, ln)],
                send_sem=send_sems.at[s],
                recv_sem=recv_sems.at[s],
                device_id=(partner,),
                device_id_type=pl.DeviceIdType.MESH,
            )
            rdma.start()
            rdma.wait()
            out_ref[pl.ds(keep_base, ln), :] = (
                out_ref[pl.ds(keep_base, ln), :]
                + rs_recv[pl.ds(rbase, ln), :].astype(jnp.float32)
            )

        z = out_ref[pl.ds(eighth, SEG), :]
        g_ref[pl.ds(eighth, SEG), :] = _gelu(z).astype(jnp.bfloat16)

        ag_steps = (
            (3, py, SEG, eighth),
            (4, px, 384, quart),
            (5, pz, 768, half),
        )
        for s, partner, ln, base in ag_steps:
            rdma = pltpu.make_async_remote_copy(
                src_ref=g_ref.at[pl.ds(base, ln)],
                dst_ref=g_ref.at[pl.ds(base, ln)],
                send_sem=send_sems.at[s],
                recv_sem=recv_sems.at[s],
                device_id=(partner,),
                device_id_type=pl.DeviceIdType.MESH,
            )
            rdma.start()
            rdma.wait()

        out_ref[:, :] = g_ref[:, :].astype(jnp.float32)

    return pl.pallas_call(
        body,
        out_shape=jax.ShapeDtypeStruct((M, N), jnp.float32),
        in_specs=[
            pl.BlockSpec(memory_space=pltpu.VMEM),
            pl.BlockSpec(memory_space=pltpu.VMEM),
        ],
        out_specs=pl.BlockSpec(memory_space=pltpu.VMEM),
        scratch_shapes=[
            pltpu.VMEM((M, N), jnp.bfloat16),
            pltpu.VMEM((1344, N), jnp.bfloat16),
            pltpu.VMEM((768, N), jnp.bfloat16),
            pltpu.SemaphoreType.DMA((6,)),
            pltpu.SemaphoreType.DMA((6,)),
        ],
        compiler_params=pltpu.CompilerParams(collective_id=0),
    )(A16, B16)


# baseline (device time: 115633 ns/iter reference)
import jax
import jax.numpy as jnp
from jax import lax
from jax.experimental import pallas as pl
from jax.experimental.pallas import tpu as pltpu

N_DEV = 8
M = 1536
N = 1536
SEG = M // N_DEV


def _gelu(z):
    return 0.5 * z * (1.0 + jnp.tanh(0.7978845608 * (z + 0.044715 * z * z * z)))


def kernel(A, B):
    A16 = A.astype(jnp.bfloat16)
    B16 = B.astype(jnp.bfloat16)

    def body(a_ref, b_ref, out_ref, g_ref, rs_recv, stage, send_sems, recv_sems):
        p = lax.axis_index("i")
        cz = (p >> 2) & 1
        cy = (p >> 1) & 1
        cx = (p & 1) ^ cy
        pz = p ^ 4
        px = (p & 4) | ((p & 3) ^ 1)
        py = (p & 4) | (3 - (p & 3))

        half = cz * 768
        quart = half + cx * 384
        eighth = quart + cy * SEG

        out_ref[:, :] = jnp.dot(
            a_ref[:, :], b_ref[:, :], preferred_element_type=jnp.float32
        )

        barrier = pltpu.get_barrier_semaphore()
        for nbr in (pz, px, py):
            pl.semaphore_signal(
                barrier, inc=1, device_id=(nbr,),
                device_id_type=pl.DeviceIdType.MESH,
            )
        pl.semaphore_wait(barrier, 3)

        rs_steps = (
            (0, pz, 768, (1 - cz) * 768, half, 0),
            (1, px, 384, half + (1 - cx) * 384, quart, 768),
            (2, py, SEG, quart + (1 - cy) * SEG, eighth, 1152),
        )
        for s, partner, ln, send_base, keep_base, rbase in rs_steps:
            stage[pl.ds(0, ln), :] = out_ref[pl.ds(send_base, ln), :].astype(
                jnp.bfloat16
            )
            rdma = pltpu.make_async_remote_copy(
                src_ref=stage.at[pl.ds(0, ln)],
                dst_ref=rs_recv.at[pl.ds(rbase, ln)],
                send_sem=send_sems.at[s],
                recv_sem=recv_sems.at[s],
                device_id=(partner,),
                device_id_type=pl.DeviceIdType.MESH,
            )
            rdma.start()
            rdma.wait()
            out_ref[pl.ds(keep_base, ln), :] = (
                out_ref[pl.ds(keep_base, ln), :]
                + rs_recv[pl.ds(rbase, ln), :].astype(jnp.float32)
            )

        z = out_ref[pl.ds(eighth, SEG), :]
        g_ref[pl.ds(eighth, SEG), :] = _gelu(z).astype(jnp.bfloat16)

        ag_steps = (
            (3, py, SEG, eighth),
            (4, px, 384, quart),
            (5, pz, 768, half),
        )
        for s, partner, ln, base in ag_steps:
            rdma = pltpu.make_async_remote_copy(
                src_ref=g_ref.at[pl.ds(base, ln)],
                dst_ref=g_ref.at[pl.ds(base, ln)],
                send_sem=send_sems.at[s],
                recv_sem=recv_sems.at[s],
                device_id=(partner,),
                device_id_type=pl.DeviceIdType.MESH,
            )
            rdma.start()
            rdma.wait()

        out_ref[:, :] = g_ref[:, :].astype(jnp.float32)

    return pl.pallas_call(
        body,
        out_shape=jax.ShapeDtypeStruct((M, N), jnp.float32),
        in_specs=[
            pl.BlockSpec(memory_space=pltpu.VMEM),
            pl.BlockSpec(memory_space=pltpu.VMEM),
        ],
        out_specs=pl.BlockSpec(memory_space=pltpu.VMEM),
        scratch_shapes=[
            pltpu.VMEM((M, N), jnp.bfloat16),
            pltpu.VMEM((1344, N), jnp.bfloat16),
            pltpu.VMEM((768, N), jnp.bfloat16),
            pltpu.SemaphoreType.DMA((6,)),
            pltpu.SemaphoreType.DMA((6,)),
        ],
        compiler_params=pltpu.CompilerParams(collective_id=0),
    )(A16, B16)


# device time: 56545 ns/iter; 2.0450x vs baseline; 2.0450x over previous
import jax
import jax.numpy as jnp
from jax import lax
from jax.experimental import pallas as pl
from jax.experimental.pallas import tpu as pltpu

N_DEV = 8
M = 1536
N = 1536
SEG = M // N_DEV
CW = N // 3
SIZES = (768, 384, 192)
RBASE = (0, 768, 1152)


def _gelu(z):
    return 0.5 * z * (1.0 + jnp.tanh(0.7978845608 * (z + 0.044715 * z * z * z)))


def _make_schedule(dims):
    bases = [0]
    rs = []
    b = 0
    for i, (bit, partner) in enumerate(dims):
        ln = SIZES[i]
        rs.append((partner, ln, b + (1 - bit) * ln, b + bit * ln))
        b = b + bit * ln
        bases.append(b)
    ag = []
    for i in range(3):
        ag.append((dims[2 - i][1], SIZES[2 - i], bases[3 - i]))
    return rs, ag, bases[3]


def kernel(A, B):
    A16 = A.astype(jnp.bfloat16)
    B16 = B.astype(jnp.bfloat16)

    def body(a_ref, b_ref, out_ref, g_ref, rs_recv, stage, send_sems, recv_sems):
        p = lax.axis_index("i")
        cz = (p >> 2) & 1
        cy = (p >> 1) & 1
        cx = (p & 1) ^ cy
        pz = p ^ 4
        px = (p & 4) | ((p & 3) ^ 1)
        py = (p & 4) | (3 - (p & 3))

        z_dim = (cz, pz)
        x_dim = (cx, px)
        y_dim = (cy, py)
        scheds = [
            _make_schedule([z_dim, x_dim, y_dim]),
            _make_schedule([x_dim, y_dim, z_dim]),
            _make_schedule([y_dim, z_dim, x_dim]),
        ]

        out_ref[:, :] = jnp.dot(
            a_ref[:, :], b_ref[:, :], preferred_element_type=jnp.float32
        )

        barrier = pltpu.get_barrier_semaphore()
        for nbr in (pz, px, py):
            pl.semaphore_signal(
                barrier, inc=1, device_id=(nbr,),
                device_id_type=pl.DeviceIdType.MESH,
            )
        pl.semaphore_wait(barrier, 3)

        for s in range(3):
            rdmas = []
            for r in range(3):
                partner, ln, send_base, keep_base = scheds[r][0][s]
                col = CW * r
                stage[pl.ds(0, ln), pl.ds(col, CW)] = out_ref[
                    pl.ds(send_base, ln), pl.ds(col, CW)
                ].astype(jnp.bfloat16)
                rdma = pltpu.make_async_remote_copy(
                    src_ref=stage.at[pl.ds(0, ln), pl.ds(col, CW)],
                    dst_ref=rs_recv.at[pl.ds(RBASE[s], ln), pl.ds(col, CW)],
                    send_sem=send_sems.at[s * 3 + r],
                    recv_sem=recv_sems.at[s * 3 + r],
                    device_id=(partner,),
                    device_id_type=pl.DeviceIdType.MESH,
                )
                rdma.start()
                rdmas.append((rdma, ln, keep_base, col))
            for rdma, ln, keep_base, col in rdmas:
                rdma.wait()
                out_ref[pl.ds(keep_base, ln), pl.ds(col, CW)] = (
                    out_ref[pl.ds(keep_base, ln), pl.ds(col, CW)]
                    + rs_recv[pl.ds(RBASE[s], ln), pl.ds(col, CW)].astype(
                        jnp.float32
                    )
                )

        for r in range(3):
            seg = scheds[r][2]
            col = CW * r
            zv = out_ref[pl.ds(seg, SEG), pl.ds(col, CW)]
            g_ref[pl.ds(seg, SEG), pl.ds(col, CW)] = _gelu(zv).astype(
                jnp.bfloat16
            )

        for s in range(3):
            rdmas = []
            for r in range(3):
                partner, ln, base = scheds[r][1][s]
                col = CW * r
                rdma = pltpu.make_async_remote_copy(
                    src_ref=g_ref.at[pl.ds(base, ln), pl.ds(col, CW)],
                    dst_ref=g_ref.at[pl.ds(base, ln), pl.ds(col, CW)],
                    send_sem=send_sems.at[9 + s * 3 + r],
                    recv_sem=recv_sems.at[9 + s * 3 + r],
                    device_id=(partner,),
                    device_id_type=pl.DeviceIdType.MESH,
                )
                rdma.start()
                rdmas.append(rdma)
            for rdma in rdmas:
                rdma.wait()

        out_ref[:, :] = g_ref[:, :].astype(jnp.float32)

    return pl.pallas_call(
        body,
        out_shape=jax.ShapeDtypeStruct((M, N), jnp.float32),
        in_specs=[
            pl.BlockSpec(memory_space=pltpu.VMEM),
            pl.BlockSpec(memory_space=pltpu.VMEM),
        ],
        out_specs=pl.BlockSpec(memory_space=pltpu.VMEM),
        scratch_shapes=[
            pltpu.VMEM((M, N), jnp.bfloat16),
            pltpu.VMEM((1344, N), jnp.bfloat16),
            pltpu.VMEM((768, N), jnp.bfloat16),
            pltpu.SemaphoreType.DMA((18,)),
            pltpu.SemaphoreType.DMA((18,)),
        ],
        compiler_params=pltpu.CompilerParams(collective_id=0),
    )(A16, B16)


# device time: 56319 ns/iter; 2.0532x vs baseline; 1.0040x over previous
import jax
import jax.numpy as jnp
from jax import lax
from jax.experimental import pallas as pl
from jax.experimental.pallas import tpu as pltpu

N_DEV = 8
M = 1536
N = 1536
SEG = M // N_DEV
CW = N // 3
SIZES = (768, 384, 192)
RBASE = (0, 768, 1152)
SBASE = (0, 768, 1152)


def _gelu(z):
    return 0.5 * z * (1.0 + jnp.tanh(0.7978845608 * (z + 0.044715 * z * z * z)))


def _make_schedule(dims):
    bases = [0]
    rs = []
    b = 0
    for i, (bit, partner) in enumerate(dims):
        ln = SIZES[i]
        rs.append((partner, ln, b + (1 - bit) * ln, b + bit * ln))
        b = b + bit * ln
        bases.append(b)
    ag = []
    for i in range(3):
        ln = SIZES[2 - i]
        my_base = bases[3 - i]
        parent = bases[2 - i]
        ag.append((dims[2 - i][1], ln, my_base, 2 * parent + ln - my_base))
    return rs, ag, bases[3]


def kernel(A, B):
    A16 = A.astype(jnp.bfloat16)
    B16 = B.astype(jnp.bfloat16)

    def body(a_ref, b_ref, out_ref, g_ref, rs_recv, stage, send_sems, recv_sems):
        p = lax.axis_index("i")
        cz = (p >> 2) & 1
        cy = (p >> 1) & 1
        cx = (p & 1) ^ cy
        pz = p ^ 4
        px = (p & 4) | ((p & 3) ^ 1)
        py = (p & 4) | (3 - (p & 3))

        scheds = [
            _make_schedule([(cz, pz), (cx, px), (cy, py)]),
            _make_schedule([(cx, px), (cy, py), (cz, pz)]),
            _make_schedule([(cy, py), (cz, pz), (cx, px)]),
        ]

        def rs_rdma(r, s, src_rows, ln):
            partner = scheds[r][0][s][0]
            col = CW * r
            return pltpu.make_async_remote_copy(
                src_ref=stage.at[src_rows, pl.ds(col, CW)],
                dst_ref=rs_recv.at[pl.ds(RBASE[s], ln), pl.ds(col, CW)],
                send_sem=send_sems.at[s * 3 + r],
                recv_sem=recv_sems.at[s * 3 + r],
                device_id=(partner,),
                device_id_type=pl.DeviceIdType.MESH,
            )

        def ag_rdma(r, s):
            partner, ln, base, _ = scheds[r][1][s]
            col = CW * r
            return pltpu.make_async_remote_copy(
                src_ref=g_ref.at[pl.ds(base, ln), pl.ds(col, CW)],
                dst_ref=g_ref.at[pl.ds(base, ln), pl.ds(col, CW)],
                send_sem=send_sems.at[9 + s * 3 + r],
                recv_sem=recv_sems.at[9 + s * 3 + r],
                device_id=(partner,),
                device_id_type=pl.DeviceIdType.MESH,
            )

        barrier = pltpu.get_barrier_semaphore()
        for nbr in (pz, px, py):
            pl.semaphore_signal(
                barrier, inc=1, device_id=(nbr,),
                device_id_type=pl.DeviceIdType.MESH,
            )
        pl.semaphore_wait(barrier, 3)

        all_rdmas = []

        rs_infl = [None, None, None]
        for r in range(3):
            col = CW * r
            out_ref[:, pl.ds(col, CW)] = jnp.dot(
                a_ref[:, :], b_ref[:, pl.ds(col, CW)],
                preferred_element_type=jnp.float32,
            )
            _, ln, send_base, _ = scheds[r][0][0]
            stage[pl.ds(SBASE[0], ln), pl.ds(col, CW)] = out_ref[
                pl.ds(send_base, ln), pl.ds(col, CW)
            ].astype(jnp.bfloat16)
            rdma = rs_rdma(r, 0, pl.ds(SBASE[0], ln), ln)
            rdma.start()
            rs_infl[r] = rdma
            all_rdmas.append(rdma)

        for s in range(2):
            for r in range(3):
                _, ln, _, keep_base = scheds[r][0][s]
                _, ln_n, send_base_n, keep_base_n = scheds[r][0][s + 1]
                col = CW * r
                rs_infl[r].wait_recv()
                r_send = RBASE[s] + (send_base_n - keep_base)
                r_keep = RBASE[s] + (keep_base_n - keep_base)
                stage[pl.ds(SBASE[s + 1], ln_n), pl.ds(col, CW)] = (
                    out_ref[pl.ds(send_base_n, ln_n), pl.ds(col, CW)]
                    + rs_recv[pl.ds(r_send, ln_n), pl.ds(col, CW)].astype(
                        jnp.float32
                    )
                ).astype(jnp.bfloat16)
                rdma = rs_rdma(r, s + 1, pl.ds(SBASE[s + 1], ln_n), ln_n)
                rdma.start()
                rs_infl[r] = rdma
                all_rdmas.append(rdma)
                out_ref[pl.ds(keep_base_n, ln_n), pl.ds(col, CW)] = (
                    out_ref[pl.ds(keep_base_n, ln_n), pl.ds(col, CW)]
                    + rs_recv[pl.ds(r_keep, ln_n), pl.ds(col, CW)].astype(
                        jnp.float32
                    )
                )

        ag_infl = [None, None, None]
        for r in range(3):
            seg = scheds[r][2]
            col = CW * r
            rs_infl[r].wait_recv()
            zv = (
                out_ref[pl.ds(seg, SEG), pl.ds(col, CW)]
                + rs_recv[pl.ds(RBASE[2], SEG), pl.ds(col, CW)].astype(
                    jnp.float32
                )
            )
            zg = _gelu(zv)
            g_ref[pl.ds(seg, SEG), pl.ds(col, CW)] = zg.astype(jnp.bfloat16)
            rdma = ag_rdma(r, 0)
            rdma.start()
            ag_infl[r] = rdma
            all_rdmas.append(rdma)
            out_ref[pl.ds(seg, SEG), pl.ds(col, CW)] = zg

        for s in range(3):
            for r in range(3):
                _, ln, _, inc_base = scheds[r][1][s]
                col = CW * r
                ag_infl[r].wait_recv()
                if s < 2:
                    rdma = ag_rdma(r, s + 1)
                    rdma.start()
                    ag_infl[r] = rdma
                    all_rdmas.append(rdma)
                out_ref[pl.ds(inc_base, ln), pl.ds(col, CW)] = g_ref[
                    pl.ds(inc_base, ln), pl.ds(col, CW)
                ].astype(jnp.float32)

        for rdma in all_rdmas:
            rdma.wait_send()

    return pl.pallas_call(
        body,
        out_shape=jax.ShapeDtypeStruct((M, N), jnp.float32),
        in_specs=[
            pl.BlockSpec(memory_space=pltpu.VMEM),
            pl.BlockSpec(memory_space=pltpu.VMEM),
        ],
        out_specs=pl.BlockSpec(memory_space=pltpu.VMEM),
        scratch_shapes=[
            pltpu.VMEM((M, N), jnp.bfloat16),
            pltpu.VMEM((1344, N), jnp.bfloat16),
            pltpu.VMEM((1344, N), jnp.bfloat16),
            pltpu.SemaphoreType.DMA((18,)),
            pltpu.SemaphoreType.DMA((18,)),
        ],
        compiler_params=pltpu.CompilerParams(collective_id=0),
    )(A16, B16)


# device time: 47646 ns/iter; 2.4269x vs baseline; 1.1820x over previous
import jax
import jax.numpy as jnp
from jax import lax
from jax.experimental import pallas as pl
from jax.experimental.pallas import tpu as pltpu

N_DEV = 8
M = 1536
N = 1536
SEG = M // N_DEV
NC = 6
CW = N // NC
SIZES = (768, 384, 192)
RBASE = (0, 768, 1152)


def _gelu(z):
    return 0.5 * z * (1.0 + jnp.tanh(0.7978845608 * (z + 0.044715 * z * z * z)))


def _make_schedule(dims):
    bases = [0]
    rs = []
    b = 0
    for i, (bit, partner) in enumerate(dims):
        ln = SIZES[i]
        rs.append((partner, ln, b + (1 - bit) * ln, b + bit * ln))
        b = b + bit * ln
        bases.append(b)
    ag = []
    for i in range(3):
        ln = SIZES[2 - i]
        my_base = bases[3 - i]
        parent = bases[2 - i]
        ag.append((dims[2 - i][1], ln, my_base, 2 * parent + ln - my_base))
    return rs, ag, bases[3]


def kernel(A, B):
    A16 = A.astype(jnp.bfloat16)
    B16 = B.astype(jnp.bfloat16)

    def body(a_ref, b_ref, out_ref, g_ref, rs_recv, stage, send_sems, recv_sems):
        p = lax.axis_index("i")
        cz = (p >> 2) & 1
        cy = (p >> 1) & 1
        cx = (p & 1) ^ cy
        pz = p ^ 4
        px = (p & 4) | ((p & 3) ^ 1)
        py = (p & 4) | (3 - (p & 3))

        orders = [
            [(cz, pz), (cx, px), (cy, py)],
            [(cx, px), (cy, py), (cz, pz)],
            [(cy, py), (cz, pz), (cx, px)],
        ]
        scheds = [_make_schedule(orders[c % 3]) for c in range(NC)]

        def rs_rdma(c, s, ln):
            partner = scheds[c][0][s][0]
            col = CW * c
            return pltpu.make_async_remote_copy(
                src_ref=stage.at[pl.ds(RBASE[s], ln), pl.ds(col, CW)],
                dst_ref=rs_recv.at[pl.ds(RBASE[s], ln), pl.ds(col, CW)],
                send_sem=send_sems.at[s * NC + c],
                recv_sem=recv_sems.at[s * NC + c],
                device_id=(partner,),
                device_id_type=pl.DeviceIdType.MESH,
            )

        def ag_rdma(c, s):
            partner, ln, base, _ = scheds[c][1][s]
            col = CW * c
            return pltpu.make_async_remote_copy(
                src_ref=g_ref.at[pl.ds(base, ln), pl.ds(col, CW)],
                dst_ref=g_ref.at[pl.ds(base, ln), pl.ds(col, CW)],
                send_sem=send_sems.at[18 + s * NC + c],
                recv_sem=recv_sems.at[18 + s * NC + c],
                device_id=(partner,),
                device_id_type=pl.DeviceIdType.MESH,
            )

        barrier = pltpu.get_barrier_semaphore()
        for nbr in (pz, px, py):
            pl.semaphore_signal(
                barrier, inc=1, device_id=(nbr,),
                device_id_type=pl.DeviceIdType.MESH,
            )
        pl.semaphore_wait(barrier, 3)

        all_rdmas = []
        rs_infl = [None] * NC

        for c in range(NC):
            _, ln, send_base, keep_base = scheds[c][0][0]
            col = CW * c
            out_ref[pl.ds(send_base, ln), pl.ds(col, CW)] = jnp.dot(
                a_ref[pl.ds(send_base, ln), :], b_ref[:, pl.ds(col, CW)],
                preferred_element_type=jnp.float32,
            )
            stage[pl.ds(RBASE[0], ln), pl.ds(col, CW)] = out_ref[
                pl.ds(send_base, ln), pl.ds(col, CW)
            ].astype(jnp.bfloat16)
            rdma = rs_rdma(c, 0, ln)
            rdma.start()
            rs_infl[c] = rdma
            all_rdmas.append(rdma)
            out_ref[pl.ds(keep_base, ln), pl.ds(col, CW)] = jnp.dot(
                a_ref[pl.ds(keep_base, ln), :], b_ref[:, pl.ds(col, CW)],
                preferred_element_type=jnp.float32,
            )

        for s in range(2):
            for c in range(NC):
                _, ln, _, keep_base = scheds[c][0][s]
                _, ln_n, send_base_n, keep_base_n = scheds[c][0][s + 1]
                col = CW * c
                rs_infl[c].wait_recv()
                r_send = RBASE[s] + (send_base_n - keep_base)
                r_keep = RBASE[s] + (keep_base_n - keep_base)
                stage[pl.ds(RBASE[s + 1], ln_n), pl.ds(col, CW)] = (
                    out_ref[pl.ds(send_base_n, ln_n), pl.ds(col, CW)]
                    + rs_recv[pl.ds(r_send, ln_n), pl.ds(col, CW)].astype(
                        jnp.float32
                    )
                ).astype(jnp.bfloat16)
                rdma = rs_rdma(c, s + 1, ln_n)
                rdma.start()
                rs_infl[c] = rdma
                all_rdmas.append(rdma)
                out_ref[pl.ds(keep_base_n, ln_n), pl.ds(col, CW)] = (
                    out_ref[pl.ds(keep_base_n, ln_n), pl.ds(col, CW)]
                    + rs_recv[pl.ds(r_keep, ln_n), pl.ds(col, CW)].astype(
                        jnp.float32
                    )
                )

        ag_infl = [None] * NC
        for c in range(NC):
            seg = scheds[c][2]
            col = CW * c
            rs_infl[c].wait_recv()
            zv = (
                out_ref[pl.ds(seg, SEG), pl.ds(col, CW)]
                + rs_recv[pl.ds(RBASE[2], SEG), pl.ds(col, CW)].astype(
                    jnp.float32
                )
            )
            zg = _gelu(zv)
            g_ref[pl.ds(seg, SEG), pl.ds(col, CW)] = zg.astype(jnp.bfloat16)
            rdma = ag_rdma(c, 0)
            rdma.start()
            ag_infl[c] = rdma
            all_rdmas.append(rdma)
            out_ref[pl.ds(seg, SEG), pl.ds(col, CW)] = zg

        for s in range(3):
            for c in range(NC):
                _, ln, _, inc_base = scheds[c][1][s]
                col = CW * c
                ag_infl[c].wait_recv()
                if s < 2:
                    rdma = ag_rdma(c, s + 1)
                    rdma.start()
                    ag_infl[c] = rdma
                    all_rdmas.append(rdma)
                out_ref[pl.ds(inc_base, ln), pl.ds(col, CW)] = g_ref[
                    pl.ds(inc_base, ln), pl.ds(col, CW)
                ].astype(jnp.float32)

        for rdma in all_rdmas:
            rdma.wait_send()

    return pl.pallas_call(
        body,
        out_shape=jax.ShapeDtypeStruct((M, N), jnp.float32),
        in_specs=[
            pl.BlockSpec(memory_space=pltpu.VMEM),
            pl.BlockSpec(memory_space=pltpu.VMEM),
        ],
        out_specs=pl.BlockSpec(memory_space=pltpu.VMEM),
        scratch_shapes=[
            pltpu.VMEM((M, N), jnp.bfloat16),
            pltpu.VMEM((1344, N), jnp.bfloat16),
            pltpu.VMEM((1344, N), jnp.bfloat16),
            pltpu.SemaphoreType.DMA((36,)),
            pltpu.SemaphoreType.DMA((36,)),
        ],
        compiler_params=pltpu.CompilerParams(collective_id=0),
    )(A16, B16)


# device time: 46004 ns/iter; 2.5135x vs baseline; 1.0357x over previous
import jax
import jax.numpy as jnp
from jax import lax
from jax.experimental import pallas as pl
from jax.experimental.pallas import tpu as pltpu

N_DEV = 8
M = 1536
N = 1536
SEG = M // N_DEV
NC = 6
CW = N // NC
SIZES = (768, 384, 192)
RBASE = (0, 768, 1152)


def _gelu(z):
    return 0.5 * z * (1.0 + jnp.tanh(0.7978845608 * (z + 0.044715 * z * z * z)))


def _make_schedule(dims):
    bases = [0]
    rs = []
    b = 0
    for i, (bit, partner) in enumerate(dims):
        ln = SIZES[i]
        rs.append((partner, ln, b + (1 - bit) * ln, b + bit * ln))
        b = b + bit * ln
        bases.append(b)
    ag = []
    for i in range(3):
        ln = SIZES[2 - i]
        my_base = bases[3 - i]
        parent = bases[2 - i]
        ag.append((dims[2 - i][1], ln, my_base, 2 * parent + ln - my_base))
    return rs, ag, bases[3]


def kernel(A, B):
    A16 = A.astype(jnp.bfloat16)
    B16 = B.astype(jnp.bfloat16)

    def body(a_ref, b_ref, out_ref, rs_recv, send_sems, recv_sems):
        p = lax.axis_index("i")
        cz = (p >> 2) & 1
        cy = (p >> 1) & 1
        cx = (p & 1) ^ cy
        pz = p ^ 4
        px = (p & 4) | ((p & 3) ^ 1)
        py = (p & 4) | (3 - (p & 3))

        orders = [
            [(cz, pz), (cx, px), (cy, py)],
            [(cx, px), (cy, py), (cz, pz)],
            [(cy, py), (cz, pz), (cx, px)],
        ]
        scheds = [_make_schedule(orders[c % 3]) for c in range(NC)]

        def rs_rdma(c, s):
            partner, ln, send_base, _ = scheds[c][0][s]
            col = CW * c
            return pltpu.make_async_remote_copy(
                src_ref=out_ref.at[pl.ds(send_base, ln), pl.ds(col, CW)],
                dst_ref=rs_recv.at[pl.ds(RBASE[s], ln), pl.ds(col, CW)],
                send_sem=send_sems.at[s * NC + c],
                recv_sem=recv_sems.at[s * NC + c],
                device_id=(partner,),
                device_id_type=pl.DeviceIdType.MESH,
            )

        def ag_rdma(c, s):
            partner, ln, base, _ = scheds[c][1][s]
            col = CW * c
            return pltpu.make_async_remote_copy(
                src_ref=out_ref.at[pl.ds(base, ln), pl.ds(col, CW)],
                dst_ref=out_ref.at[pl.ds(base, ln), pl.ds(col, CW)],
                send_sem=send_sems.at[18 + s * NC + c],
                recv_sem=recv_sems.at[18 + s * NC + c],
                device_id=(partner,),
                device_id_type=pl.DeviceIdType.MESH,
            )

        barrier = pltpu.get_barrier_semaphore()
        for nbr in (pz, px, py):
            pl.semaphore_signal(
                barrier, inc=1, device_id=(nbr,),
                device_id_type=pl.DeviceIdType.MESH,
            )
        pl.semaphore_wait(barrier, 3)

        all_rdmas = []
        rs_infl = [None] * NC

        for c in range(NC):
            _, ln, send_base, keep_base = scheds[c][0][0]
            col = CW * c
            out_ref[pl.ds(send_base, ln), pl.ds(col, CW)] = jnp.dot(
                a_ref[pl.ds(send_base, ln), :], b_ref[:, pl.ds(col, CW)],
                preferred_element_type=jnp.float32,
            ).astype(jnp.bfloat16)
            rdma = rs_rdma(c, 0)
            rdma.start()
            rs_infl[c] = rdma
            all_rdmas.append(rdma)
            out_ref[pl.ds(keep_base, ln), pl.ds(col, CW)] = jnp.dot(
                a_ref[pl.ds(keep_base, ln), :], b_ref[:, pl.ds(col, CW)],
                preferred_element_type=jnp.float32,
            ).astype(jnp.bfloat16)

        for s in range(2):
            for c in range(NC):
                _, ln, _, keep_base = scheds[c][0][s]
                _, ln_n, send_base_n, keep_base_n = scheds[c][0][s + 1]
                col = CW * c
                rs_infl[c].wait_recv()
                r_send = RBASE[s] + (send_base_n - keep_base)
                r_keep = RBASE[s] + (keep_base_n - keep_base)
                out_ref[pl.ds(send_base_n, ln_n), pl.ds(col, CW)] = (
                    out_ref[pl.ds(send_base_n, ln_n), pl.ds(col, CW)]
                    + rs_recv[pl.ds(r_send, ln_n), pl.ds(col, CW)]
                )
                rdma = rs_rdma(c, s + 1)
                rdma.start()
                rs_infl[c] = rdma
                all_rdmas.append(rdma)
                out_ref[pl.ds(keep_base_n, ln_n), pl.ds(col, CW)] = (
                    out_ref[pl.ds(keep_base_n, ln_n), pl.ds(col, CW)]
                    + rs_recv[pl.ds(r_keep, ln_n), pl.ds(col, CW)]
                )

        ag_infl = [None] * NC
        for c in range(NC):
            seg = scheds[c][2]
            col = CW * c
            rs_infl[c].wait_recv()
            zv = (
                out_ref[pl.ds(seg, SEG), pl.ds(col, CW)].astype(jnp.float32)
                + rs_recv[pl.ds(RBASE[2], SEG), pl.ds(col, CW)].astype(
                    jnp.float32
                )
            )
            out_ref[pl.ds(seg, SEG), pl.ds(col, CW)] = _gelu(zv).astype(
                jnp.bfloat16
            )
            rdma = ag_rdma(c, 0)
            rdma.start()
            ag_infl[c] = rdma
            all_rdmas.append(rdma)

        for s in range(3):
            for c in range(NC):
                ag_infl[c].wait_recv()
                if s < 2:
                    rdma = ag_rdma(c, s + 1)
                    rdma.start()
                    ag_infl[c] = rdma
                    all_rdmas.append(rdma)

        for rdma in all_rdmas:
            rdma.wait_send()

    return pl.pallas_call(
        body,
        out_shape=jax.ShapeDtypeStruct((M, N), jnp.bfloat16),
        in_specs=[
            pl.BlockSpec(memory_space=pltpu.VMEM),
            pl.BlockSpec(memory_space=pltpu.VMEM),
        ],
        out_specs=pl.BlockSpec(memory_space=pltpu.VMEM),
        scratch_shapes=[
            pltpu.VMEM((1344, N), jnp.bfloat16),
            pltpu.SemaphoreType.DMA((36,)),
            pltpu.SemaphoreType.DMA((36,)),
        ],
        compiler_params=pltpu.CompilerParams(collective_id=0),
    )(A16, B16)
